# baseline (speedup 1.0000x reference)
import functools

import jax
import jax.numpy as jnp
from jax.experimental import pallas as pl

D_MODEL = 2048
N_EXP = 16
TOP_K = 2
TILE = 512


def _router_kernel(h_ref, w_ref, mask_ref, probs_ref, logits_ref):
    h = h_ref[...]
    w = w_ref[...]
    logits = jax.lax.dot_general(
        h, w, (((1,), (1,)), ((), ())), preferred_element_type=jnp.float32
    )

    m = jnp.max(logits, axis=-1, keepdims=True)
    e = jnp.exp(logits - m)
    probs = e / jnp.sum(e, axis=-1, keepdims=True)

    col = jax.lax.broadcasted_iota(jnp.int32, logits.shape, 1)
    m1 = jnp.max(logits, axis=-1, keepdims=True)
    idx1 = jnp.min(jnp.where(logits == m1, col, N_EXP), axis=-1, keepdims=True)
    mask1 = col == idx1
    l2 = jnp.where(mask1, -jnp.inf, logits)
    m2 = jnp.max(l2, axis=-1, keepdims=True)
    idx2 = jnp.min(jnp.where(l2 == m2, col, N_EXP), axis=-1, keepdims=True)
    mask = mask1 | (col == idx2)

    mask_ref[...] = mask
    probs_ref[...] = probs
    logits_ref[...] = logits


@functools.partial(jax.jit, static_argnames=())
def kernel(h, W):
    n_tok = h.shape[0]
    grid = (n_tok // TILE,)
    out_shapes = (
        jax.ShapeDtypeStruct((n_tok, N_EXP), jnp.bool_),
        jax.ShapeDtypeStruct((n_tok, N_EXP), jnp.float32),
        jax.ShapeDtypeStruct((n_tok, N_EXP), jnp.float32),
    )
    out_spec = pl.BlockSpec((TILE, N_EXP), lambda i: (i, 0))
    mask, probs, logits = pl.pallas_call(
        _router_kernel,
        grid=grid,
        in_specs=[
            pl.BlockSpec((TILE, D_MODEL), lambda i: (i, 0)),
            pl.BlockSpec((N_EXP, D_MODEL), lambda i: (0, 0)),
        ],
        out_specs=(out_spec, out_spec, out_spec),
        out_shape=out_shapes,
    )(h, W)
    return mask, probs, logits


# speedup vs baseline: 1.1112x; 1.1112x over previous
import functools

import jax
import jax.numpy as jnp
from jax.experimental import pallas as pl

D_MODEL = 2048
N_EXP = 16
TOP_K = 2
TILE = 1024


def _router_kernel(h_ref, w_ref, mask_ref, probs_ref, logits_ref):
    h = h_ref[...]
    w = w_ref[...]
    logits = jax.lax.dot_general(
        h, w, (((1,), (1,)), ((), ())), preferred_element_type=jnp.float32
    )

    m = jnp.max(logits, axis=-1, keepdims=True)
    e = jnp.exp(logits - m)
    probs = e / jnp.sum(e, axis=-1, keepdims=True)

    col = jax.lax.broadcasted_iota(jnp.int32, logits.shape, 1)
    m1 = jnp.max(logits, axis=-1, keepdims=True)
    idx1 = jnp.min(jnp.where(logits == m1, col, N_EXP), axis=-1, keepdims=True)
    mask1 = col == idx1
    l2 = jnp.where(mask1, -jnp.inf, logits)
    m2 = jnp.max(l2, axis=-1, keepdims=True)
    idx2 = jnp.min(jnp.where(l2 == m2, col, N_EXP), axis=-1, keepdims=True)
    mask = mask1 | (col == idx2)

    mask_ref[...] = mask
    probs_ref[...] = probs
    logits_ref[...] = logits


@functools.partial(jax.jit, static_argnames=())
def kernel(h, W):
    n_tok = h.shape[0]
    grid = (n_tok // TILE,)
    out_shapes = (
        jax.ShapeDtypeStruct((n_tok, N_EXP), jnp.bool_),
        jax.ShapeDtypeStruct((n_tok, N_EXP), jnp.float32),
        jax.ShapeDtypeStruct((n_tok, N_EXP), jnp.float32),
    )
    out_spec = pl.BlockSpec((TILE, N_EXP), lambda i: (i, 0))
    mask, probs, logits = pl.pallas_call(
        _router_kernel,
        grid=grid,
        in_specs=[
            pl.BlockSpec((TILE, D_MODEL), lambda i: (i, 0)),
            pl.BlockSpec((N_EXP, D_MODEL), lambda i: (0, 0)),
        ],
        out_specs=(out_spec, out_spec, out_spec),
        out_shape=out_shapes,
    )(h, W)
    return mask, probs, logits


# speedup vs baseline: 1.7553x; 1.5797x over previous
import functools

import jax
import jax.numpy as jnp
from jax.experimental import pallas as pl
from jax.experimental.pallas import tpu as pltpu

D_MODEL = 2048
N_EXP = 16
TOP_K = 2
TILE = 1024


def _router_kernel(h_ref, w_ref, mask_ref, probs_ref, logits_ref):
    h = h_ref[...]
    w = w_ref[...]
    logits = jax.lax.dot_general(
        w, h, (((1,), (1,)), ((), ())), preferred_element_type=jnp.float32
    )

    m = jnp.max(logits, axis=0, keepdims=True)
    e = jnp.exp(logits - m)
    probs = e / jnp.sum(e, axis=0, keepdims=True)

    row = jax.lax.broadcasted_iota(jnp.int32, logits.shape, 0)
    idx1 = jnp.min(jnp.where(logits == m, row, N_EXP), axis=0, keepdims=True)
    mask1 = row == idx1
    l2 = jnp.where(mask1, -jnp.inf, logits)
    m2 = jnp.max(l2, axis=0, keepdims=True)
    idx2 = jnp.min(jnp.where(l2 == m2, row, N_EXP), axis=0, keepdims=True)
    mask = mask1 | (row == idx2)

    mask_ref[...] = mask
    probs_ref[...] = probs
    logits_ref[...] = logits


@functools.partial(jax.jit, static_argnames=())
def kernel(h, W):
    n_tok = h.shape[0]
    grid = (n_tok // TILE,)
    out_shapes = (
        jax.ShapeDtypeStruct((N_EXP, n_tok), jnp.bool_),
        jax.ShapeDtypeStruct((N_EXP, n_tok), jnp.float32),
        jax.ShapeDtypeStruct((N_EXP, n_tok), jnp.float32),
    )
    out_spec = pl.BlockSpec((N_EXP, TILE), lambda i: (0, i))
    mask_t, probs_t, logits_t = pl.pallas_call(
        _router_kernel,
        grid=grid,
        in_specs=[
            pl.BlockSpec((TILE, D_MODEL), lambda i: (i, 0)),
            pl.BlockSpec((N_EXP, D_MODEL), lambda i: (0, 0)),
        ],
        out_specs=(out_spec, out_spec, out_spec),
        out_shape=out_shapes,
        compiler_params=pltpu.CompilerParams(
            dimension_semantics=("parallel",),
        ),
    )(h, W)
    return mask_t.T, probs_t.T, logits_t.T
